# Initial kernel scaffold; baseline (speedup 1.0000x reference)
import jax
import jax.numpy as jnp
from jax.experimental import pallas as pl


def kernel(pcl_feat, joint_feat, pcl_xyz, joint_xyz, W_rel, b_rel, W_mlp, b_mlp, gamma, beta):
    raise NotImplementedError("write your pallas kernel here")


# speedup vs baseline: 32.0813x; 32.0813x over previous
import functools

import jax
import jax.numpy as jnp
from jax.experimental import pallas as pl
from jax.experimental.pallas import tpu as pltpu

B, N, J, D = 16, 4096, 21, 256
JP = 128
TOPK = 4
BLK = 1024
NB = N // BLK
COUNT = float(B * N)


def _main_kernel(xyz1_ref, xyz2t_ref, pcl_ref, jf_ref, rel_ref, bmlp_ref,
                 w1t_ref, w2t_ref, out_ref, stats_ref):
    first = (pl.program_id(0) == 0) & (pl.program_id(1) == 0)

    x1 = xyz1_ref[0]
    x2t = xyz2t_ref[0]
    w_d = rel_ref[0]
    b_rel = rel_ref[4]
    iota = jax.lax.broadcasted_iota(jnp.int32, (BLK, JP), 1)
    valid = iota < J
    dot = jnp.dot(x1.astype(jnp.bfloat16), x2t.astype(jnp.bfloat16),
                  preferred_element_type=jnp.float32)
    x1sq = jnp.zeros((BLK, 1), dtype=jnp.float32)
    x2sq = jnp.zeros((1, JP), dtype=jnp.float32)
    x1w = jnp.zeros((BLK, 1), dtype=jnp.float32)
    s2 = jnp.zeros((1, JP), dtype=jnp.float32)
    for c in range(3):
        x1c = x1[:, c:c + 1]
        x2c = x2t[c:c + 1, :]
        x1sq = x1sq + x1c * x1c
        x2sq = x2sq + x2c * x2c
        wc = rel_ref[c + 1]
        x1w = x1w + x1c * wc
        s2 = s2 + x2c * wc
    dist = -2.0 * dot
    dist = dist + x1sq
    dist = dist + x2sq

    a_mat = jnp.zeros((BLK, JP), dtype=jnp.float32)
    d_work = jnp.where(valid, dist, jnp.inf)
    for _ in range(TOPK):
        m = jnp.min(d_work, axis=1, keepdims=True)
        is_min = d_work == m
        idxv = jnp.min(jnp.where(is_min, iota, JP), axis=1, keepdims=True)
        onehot = (iota == idxv).astype(jnp.float32)
        s2g = jnp.sum(onehot * s2, axis=1, keepdims=True)
        w = jax.nn.sigmoid(w_d * m + x1w - s2g + b_rel)
        a_mat = a_mat + onehot * (w * 0.25)
        d_work = jnp.where(onehot > 0.0, jnp.inf, d_work)

    p2 = jnp.dot(jf_ref[0], w2t_ref[...],
                 preferred_element_type=jnp.float32)
    x = jnp.dot(pcl_ref[0], w1t_ref[...],
                preferred_element_type=jnp.float32)
    x = x + jnp.dot(a_mat, p2, preferred_element_type=jnp.float32)
    x = x + bmlp_ref[0:1, :]
    out_ref[0] = x

    ssum = jnp.sum(x, axis=0, keepdims=True)
    ssq = jnp.sum(x * x, axis=0, keepdims=True)

    @pl.when(first)
    def _():
        stats_ref[...] = jnp.zeros_like(stats_ref)

    stats_ref[0:1, :] += ssum
    stats_ref[1:2, :] += ssq


def _bn_kernel(x_ref, stats_ref, gb_ref, out_ref):
    mean = stats_ref[0:1, :] * (1.0 / COUNT)
    ex2 = stats_ref[1:2, :] * (1.0 / COUNT)
    var = ex2 - mean * mean
    inv = jax.lax.rsqrt(var + 1e-5)
    scale = gb_ref[0:1, :] * inv
    shift = gb_ref[1:2, :] - mean * scale
    out_ref[0] = jnp.maximum(x_ref[0] * scale + shift, 0.0)


@functools.partial(jax.jit)
def kernel(pcl_feat, joint_feat, pcl_xyz, joint_xyz, W_rel, b_rel,
           W_mlp, b_mlp, gamma, beta):
    xyz2t = jnp.pad(joint_xyz.transpose(0, 2, 1),
                    ((0, 0), (0, 0), (0, JP - J)))
    jf_p = jnp.pad(joint_feat, ((0, 0), (0, JP - J), (0, 0)))
    rel = jnp.concatenate([W_rel, b_rel.reshape(1)])
    w1t = W_mlp[:, :D].T
    w2t = W_mlp[:, D:].T
    bmlp = b_mlp.reshape(1, D)
    gb = jnp.stack([gamma, beta])

    x, stats = pl.pallas_call(
        _main_kernel,
        grid=(B, NB),
        in_specs=[
            pl.BlockSpec((1, BLK, 3), lambda b, n: (b, n, 0)),
            pl.BlockSpec((1, 3, JP), lambda b, n: (b, 0, 0)),
            pl.BlockSpec((1, BLK, D), lambda b, n: (b, n, 0)),
            pl.BlockSpec((1, JP, D), lambda b, n: (b, 0, 0)),
            pl.BlockSpec(memory_space=pltpu.SMEM),
            pl.BlockSpec((1, D), lambda b, n: (0, 0)),
            pl.BlockSpec((D, D), lambda b, n: (0, 0)),
            pl.BlockSpec((D, D), lambda b, n: (0, 0)),
        ],
        out_specs=[
            pl.BlockSpec((1, BLK, D), lambda b, n: (b, n, 0)),
            pl.BlockSpec((8, D), lambda b, n: (0, 0)),
        ],
        out_shape=[
            jax.ShapeDtypeStruct((B, N, D), jnp.float32),
            jax.ShapeDtypeStruct((8, D), jnp.float32),
        ],
    )(pcl_xyz, xyz2t, pcl_feat, jf_p, rel, bmlp, w1t, w2t)

    y = pl.pallas_call(
        _bn_kernel,
        grid=(B,),
        in_specs=[
            pl.BlockSpec((1, N, D), lambda b: (b, 0, 0)),
            pl.BlockSpec((8, D), lambda b: (0, 0)),
            pl.BlockSpec((2, D), lambda b: (0, 0)),
        ],
        out_specs=pl.BlockSpec((1, N, D), lambda b: (b, 0, 0)),
        out_shape=jax.ShapeDtypeStruct((B, N, D), jnp.float32),
    )(x, stats, gb)
    return y


# speedup vs baseline: 40.9350x; 1.2760x over previous
import functools

import jax
import jax.numpy as jnp
from jax.experimental import pallas as pl
from jax.experimental.pallas import tpu as pltpu

B, N, J, D = 16, 4096, 21, 256
JP = 128
TOPK = 4
BLK = 1024
NB = N // BLK
COUNT = float(B * N)


def _main_kernel(xyz1_ref, xyz2t_ref, pcl_ref, jf_ref, rel_ref, bmlp_ref,
                 w1t_ref, w2t_ref, out_ref, stats_ref):
    first = (pl.program_id(0) == 0) & (pl.program_id(1) == 0)

    x1 = xyz1_ref[0]
    x2t = xyz2t_ref[0]
    w_d = rel_ref[0]
    b_rel = rel_ref[4]
    iota = jax.lax.broadcasted_iota(jnp.int32, (BLK, JP), 1)
    valid = iota < J
    dot = jnp.dot(x1.astype(jnp.bfloat16), x2t.astype(jnp.bfloat16),
                  preferred_element_type=jnp.float32)
    x1sq = jnp.zeros((BLK, 1), dtype=jnp.float32)
    x2sq = jnp.zeros((1, JP), dtype=jnp.float32)
    x1w = jnp.zeros((BLK, 1), dtype=jnp.float32)
    s2 = jnp.zeros((1, JP), dtype=jnp.float32)
    for c in range(3):
        x1c = x1[:, c:c + 1]
        x2c = x2t[c:c + 1, :]
        x1sq = x1sq + x1c * x1c
        x2sq = x2sq + x2c * x2c
        wc = rel_ref[c + 1]
        x1w = x1w + x1c * wc
        s2 = s2 + x2c * wc
    dist = -2.0 * dot
    dist = dist + x1sq
    dist = dist + x2sq

    a_mat = jnp.zeros((BLK, JP), dtype=jnp.float32)
    d_work = jnp.where(valid, dist, jnp.inf)
    for _ in range(TOPK):
        m = jnp.min(d_work, axis=1, keepdims=True)
        is_min = d_work == m
        idxv = jnp.min(jnp.where(is_min, iota, JP), axis=1, keepdims=True)
        onehot = (iota == idxv).astype(jnp.float32)
        s2g = jnp.sum(onehot * s2, axis=1, keepdims=True)
        w = jax.nn.sigmoid(w_d * m + x1w - s2g + b_rel)
        a_mat = a_mat + onehot * (w * 0.25)
        d_work = jnp.where(onehot > 0.0, jnp.inf, d_work)

    p2 = jnp.dot(jf_ref[0].astype(jnp.bfloat16), w2t_ref[...].astype(jnp.bfloat16),
                 preferred_element_type=jnp.float32)
    x = jnp.dot(pcl_ref[0].astype(jnp.bfloat16), w1t_ref[...].astype(jnp.bfloat16),
                preferred_element_type=jnp.float32)
    x = x + jnp.dot(a_mat.astype(jnp.bfloat16), p2.astype(jnp.bfloat16),
                    preferred_element_type=jnp.float32)
    x = x + bmlp_ref[0:1, :]
    out_ref[0] = x

    ssum = jnp.sum(x, axis=0, keepdims=True)
    ssq = jnp.sum(x * x, axis=0, keepdims=True)

    @pl.when(first)
    def _():
        stats_ref[...] = jnp.zeros_like(stats_ref)

    stats_ref[0:1, :] += ssum
    stats_ref[1:2, :] += ssq


def _bn_kernel(x_ref, stats_ref, gb_ref, out_ref):
    mean = stats_ref[0:1, :] * (1.0 / COUNT)
    ex2 = stats_ref[1:2, :] * (1.0 / COUNT)
    var = ex2 - mean * mean
    inv = jax.lax.rsqrt(var + 1e-5)
    scale = gb_ref[0:1, :] * inv
    shift = gb_ref[1:2, :] - mean * scale
    out_ref[0] = jnp.maximum(x_ref[0] * scale + shift, 0.0)


@functools.partial(jax.jit)
def kernel(pcl_feat, joint_feat, pcl_xyz, joint_xyz, W_rel, b_rel,
           W_mlp, b_mlp, gamma, beta):
    xyz2t = jnp.pad(joint_xyz.transpose(0, 2, 1),
                    ((0, 0), (0, 0), (0, JP - J)))
    jf_p = jnp.pad(joint_feat, ((0, 0), (0, JP - J), (0, 0)))
    rel = jnp.concatenate([W_rel, b_rel.reshape(1)])
    w1t = W_mlp[:, :D].T
    w2t = W_mlp[:, D:].T
    bmlp = b_mlp.reshape(1, D)
    gb = jnp.stack([gamma, beta])

    x, stats = pl.pallas_call(
        _main_kernel,
        grid=(B, NB),
        in_specs=[
            pl.BlockSpec((1, BLK, 3), lambda b, n: (b, n, 0)),
            pl.BlockSpec((1, 3, JP), lambda b, n: (b, 0, 0)),
            pl.BlockSpec((1, BLK, D), lambda b, n: (b, n, 0)),
            pl.BlockSpec((1, JP, D), lambda b, n: (b, 0, 0)),
            pl.BlockSpec(memory_space=pltpu.SMEM),
            pl.BlockSpec((1, D), lambda b, n: (0, 0)),
            pl.BlockSpec((D, D), lambda b, n: (0, 0)),
            pl.BlockSpec((D, D), lambda b, n: (0, 0)),
        ],
        out_specs=[
            pl.BlockSpec((1, BLK, D), lambda b, n: (b, n, 0)),
            pl.BlockSpec((8, D), lambda b, n: (0, 0)),
        ],
        out_shape=[
            jax.ShapeDtypeStruct((B, N, D), jnp.float32),
            jax.ShapeDtypeStruct((8, D), jnp.float32),
        ],
    )(pcl_xyz, xyz2t, pcl_feat, jf_p, rel, bmlp, w1t, w2t)

    y = pl.pallas_call(
        _bn_kernel,
        grid=(B,),
        in_specs=[
            pl.BlockSpec((1, N, D), lambda b: (b, 0, 0)),
            pl.BlockSpec((8, D), lambda b: (0, 0)),
            pl.BlockSpec((2, D), lambda b: (0, 0)),
        ],
        out_specs=pl.BlockSpec((1, N, D), lambda b: (b, 0, 0)),
        out_shape=jax.ShapeDtypeStruct((B, N, D), jnp.float32),
    )(x, stats, gb)
    return y


# speedup vs baseline: 104.2411x; 2.5465x over previous
import functools

import jax
import jax.numpy as jnp
from jax.experimental import pallas as pl
from jax.experimental.pallas import tpu as pltpu

B, N, J, D = 16, 4096, 21, 256
JP = 32
TOPK = 4
BLK = 1024
NB = N // BLK
COUNT = float(B * N)


def _main_kernel(x1t_ref, x2p_ref, pcl_ref, jf_ref, rel_ref, bmlp_ref,
                 w1t_ref, w2t_ref, out_ref, stats_ref):
    first = (pl.program_id(0) == 0) & (pl.program_id(1) == 0)

    x2 = x2p_ref[0]
    x1t = x1t_ref[0]
    w_d = rel_ref[0]
    b_rel = rel_ref[4]
    dot = jnp.dot(x2.astype(jnp.bfloat16), x1t.astype(jnp.bfloat16),
                  preferred_element_type=jnp.float32)
    x1sq = jnp.zeros((1, BLK), dtype=jnp.float32)
    x2sq = jnp.zeros((JP, 1), dtype=jnp.float32)
    x1w = jnp.zeros((1, BLK), dtype=jnp.float32)
    s2 = jnp.zeros((JP, 1), dtype=jnp.float32)
    for c in range(3):
        x1c = x1t[c:c + 1, :]
        x2c = x2[:, c:c + 1]
        x1sq = x1sq + x1c * x1c
        x2sq = x2sq + x2c * x2c
        wc = rel_ref[c + 1]
        x1w = x1w + x1c * wc
        s2 = s2 + x2c * wc
    dist = -2.0 * dot
    dist = dist + x1sq
    dist = dist + x2sq

    afac = jax.nn.sigmoid(w_d * dist + x1w + (b_rel - s2)) * 0.25

    iota = jax.lax.broadcasted_iota(jnp.int32, (JP, BLK), 0)
    valid = iota < J
    a_mat = jnp.zeros((JP, BLK), dtype=jnp.float32)
    d_work = jnp.where(valid, dist, jnp.inf)
    for _ in range(TOPK):
        m = jnp.min(d_work, axis=0, keepdims=True)
        is_min = d_work == m
        idxv = jnp.min(jnp.where(is_min, iota, JP), axis=0, keepdims=True)
        onehot = iota == idxv
        a_mat = jnp.where(onehot, afac, a_mat)
        d_work = jnp.where(onehot, jnp.inf, d_work)

    p2 = jnp.dot(jf_ref[0].astype(jnp.bfloat16), w2t_ref[...].astype(jnp.bfloat16),
                 preferred_element_type=jnp.float32)
    x = jnp.dot(pcl_ref[0].astype(jnp.bfloat16), w1t_ref[...].astype(jnp.bfloat16),
                preferred_element_type=jnp.float32)
    x = x + jax.lax.dot_general(
        a_mat.astype(jnp.bfloat16), p2.astype(jnp.bfloat16),
        (((0,), (0,)), ((), ())),
        preferred_element_type=jnp.float32)
    x = x + bmlp_ref[0:1, :]
    out_ref[0] = x

    ssum = jnp.sum(x, axis=0, keepdims=True)
    ssq = jnp.sum(x * x, axis=0, keepdims=True)

    @pl.when(first)
    def _():
        stats_ref[...] = jnp.zeros_like(stats_ref)

    stats_ref[0:1, :] += ssum
    stats_ref[1:2, :] += ssq


def _bn_kernel(x_ref, stats_ref, gb_ref, out_ref):
    mean = stats_ref[0:1, :] * (1.0 / COUNT)
    ex2 = stats_ref[1:2, :] * (1.0 / COUNT)
    var = ex2 - mean * mean
    inv = jax.lax.rsqrt(var + 1e-5)
    scale = gb_ref[0:1, :] * inv
    shift = gb_ref[1:2, :] - mean * scale
    out_ref[0] = jnp.maximum(x_ref[0] * scale + shift, 0.0)


@functools.partial(jax.jit)
def kernel(pcl_feat, joint_feat, pcl_xyz, joint_xyz, W_rel, b_rel,
           W_mlp, b_mlp, gamma, beta):
    x1t = pcl_xyz.transpose(0, 2, 1)
    x2p = jnp.pad(joint_xyz, ((0, 0), (0, JP - J), (0, 0)))
    jf_p = jnp.pad(joint_feat, ((0, 0), (0, JP - J), (0, 0)))
    rel = jnp.concatenate([W_rel, b_rel.reshape(1)])
    w1t = W_mlp[:, :D].T
    w2t = W_mlp[:, D:].T
    bmlp = b_mlp.reshape(1, D)
    gb = jnp.stack([gamma, beta])

    x, stats = pl.pallas_call(
        _main_kernel,
        grid=(B, NB),
        in_specs=[
            pl.BlockSpec((1, 3, BLK), lambda b, n: (b, 0, n)),
            pl.BlockSpec((1, JP, 3), lambda b, n: (b, 0, 0)),
            pl.BlockSpec((1, BLK, D), lambda b, n: (b, n, 0)),
            pl.BlockSpec((1, JP, D), lambda b, n: (b, 0, 0)),
            pl.BlockSpec(memory_space=pltpu.SMEM),
            pl.BlockSpec((1, D), lambda b, n: (0, 0)),
            pl.BlockSpec((D, D), lambda b, n: (0, 0)),
            pl.BlockSpec((D, D), lambda b, n: (0, 0)),
        ],
        out_specs=[
            pl.BlockSpec((1, BLK, D), lambda b, n: (b, n, 0)),
            pl.BlockSpec((8, D), lambda b, n: (0, 0)),
        ],
        out_shape=[
            jax.ShapeDtypeStruct((B, N, D), jnp.float32),
            jax.ShapeDtypeStruct((8, D), jnp.float32),
        ],
    )(x1t, x2p, pcl_feat, jf_p, rel, bmlp, w1t, w2t)

    y = pl.pallas_call(
        _bn_kernel,
        grid=(B,),
        in_specs=[
            pl.BlockSpec((1, N, D), lambda b: (b, 0, 0)),
            pl.BlockSpec((8, D), lambda b: (0, 0)),
            pl.BlockSpec((2, D), lambda b: (0, 0)),
        ],
        out_specs=pl.BlockSpec((1, N, D), lambda b: (b, 0, 0)),
        out_shape=jax.ShapeDtypeStruct((B, N, D), jnp.float32),
    )(x, stats, gb)
    return y


# speedup vs baseline: 124.4310x; 1.1937x over previous
import functools

import jax
import jax.numpy as jnp
from jax.experimental import pallas as pl
from jax.experimental.pallas import tpu as pltpu

B, N, J, D = 16, 4096, 21, 256
JP = 32
TOPK = 4
BLK = 2048
NB = N // BLK
COUNT = float(B * N)


def _main_kernel(x1t_ref, x2p_ref, pcl_ref, jf_ref, rel_ref, bmlp_ref,
                 w1t_ref, w2t_ref, out_ref, stats_ref):
    first = (pl.program_id(0) == 0) & (pl.program_id(1) == 0)

    x2 = x2p_ref[0]
    x1t = x1t_ref[0]
    w_d = rel_ref[0]
    b_rel = rel_ref[4]
    dot = jnp.dot(x2.astype(jnp.bfloat16), x1t.astype(jnp.bfloat16),
                  preferred_element_type=jnp.float32)
    x1sq = jnp.zeros((1, BLK), dtype=jnp.float32)
    x2sq = jnp.zeros((JP, 1), dtype=jnp.float32)
    x1w = jnp.zeros((1, BLK), dtype=jnp.float32)
    s2 = jnp.zeros((JP, 1), dtype=jnp.float32)
    for c in range(3):
        x1c = x1t[c:c + 1, :]
        x2c = x2[:, c:c + 1]
        x1sq = x1sq + x1c * x1c
        x2sq = x2sq + x2c * x2c
        wc = rel_ref[c + 1]
        x1w = x1w + x1c * wc
        s2 = s2 + x2c * wc
    dist = -2.0 * dot
    dist = dist + x1sq
    dist = dist + x2sq

    afac = (jax.nn.sigmoid(w_d * dist + x1w + (b_rel - s2)) * 0.25
            ).astype(jnp.bfloat16)

    iota = jax.lax.broadcasted_iota(jnp.int32, (JP, BLK), 0)
    valid = iota < J
    a_mat = jnp.zeros((JP, BLK), dtype=jnp.bfloat16)
    d_work = jnp.where(valid, dist, jnp.inf)
    for _ in range(TOPK):
        m = jnp.min(d_work, axis=0, keepdims=True)
        is_min = d_work == m
        idxv = jnp.min(jnp.where(is_min, iota, JP), axis=0, keepdims=True)
        onehot = iota == idxv
        a_mat = jnp.where(onehot, afac, a_mat)
        d_work = jnp.where(onehot, jnp.inf, d_work)

    p2 = jnp.dot(jf_ref[0].astype(jnp.bfloat16), w2t_ref[...].astype(jnp.bfloat16),
                 preferred_element_type=jnp.float32)
    x = jnp.dot(pcl_ref[0].astype(jnp.bfloat16), w1t_ref[...].astype(jnp.bfloat16),
                preferred_element_type=jnp.float32)
    x = x + jax.lax.dot_general(
        a_mat, p2.astype(jnp.bfloat16),
        (((0,), (0,)), ((), ())),
        preferred_element_type=jnp.float32)
    x = x + bmlp_ref[0:1, :]
    out_ref[0] = x

    ssum = jnp.sum(x, axis=0, keepdims=True)
    ssq = jnp.sum(x * x, axis=0, keepdims=True)

    @pl.when(first)
    def _():
        stats_ref[...] = jnp.zeros_like(stats_ref)

    stats_ref[0:1, :] += ssum
    stats_ref[1:2, :] += ssq


def _bn_kernel(x_ref, stats_ref, gb_ref, out_ref):
    mean = stats_ref[0:1, :] * (1.0 / COUNT)
    ex2 = stats_ref[1:2, :] * (1.0 / COUNT)
    var = ex2 - mean * mean
    inv = jax.lax.rsqrt(var + 1e-5)
    scale = gb_ref[0:1, :] * inv
    shift = gb_ref[1:2, :] - mean * scale
    out_ref[0] = jnp.maximum(x_ref[0] * scale + shift, 0.0)


@functools.partial(jax.jit)
def kernel(pcl_feat, joint_feat, pcl_xyz, joint_xyz, W_rel, b_rel,
           W_mlp, b_mlp, gamma, beta):
    x1t = pcl_xyz.transpose(0, 2, 1)
    x2p = jnp.pad(joint_xyz, ((0, 0), (0, JP - J), (0, 0)))
    jf_p = jnp.pad(joint_feat, ((0, 0), (0, JP - J), (0, 0)))
    rel = jnp.concatenate([W_rel, b_rel.reshape(1)])
    w1t = W_mlp[:, :D].T
    w2t = W_mlp[:, D:].T
    bmlp = b_mlp.reshape(1, D)
    gb = jnp.stack([gamma, beta])

    x, stats = pl.pallas_call(
        _main_kernel,
        grid=(B, NB),
        in_specs=[
            pl.BlockSpec((1, 3, BLK), lambda b, n: (b, 0, n)),
            pl.BlockSpec((1, JP, 3), lambda b, n: (b, 0, 0)),
            pl.BlockSpec((1, BLK, D), lambda b, n: (b, n, 0)),
            pl.BlockSpec((1, JP, D), lambda b, n: (b, 0, 0)),
            pl.BlockSpec(memory_space=pltpu.SMEM),
            pl.BlockSpec((1, D), lambda b, n: (0, 0)),
            pl.BlockSpec((D, D), lambda b, n: (0, 0)),
            pl.BlockSpec((D, D), lambda b, n: (0, 0)),
        ],
        out_specs=[
            pl.BlockSpec((1, BLK, D), lambda b, n: (b, n, 0)),
            pl.BlockSpec((8, D), lambda b, n: (0, 0)),
        ],
        out_shape=[
            jax.ShapeDtypeStruct((B, N, D), jnp.float32),
            jax.ShapeDtypeStruct((8, D), jnp.float32),
        ],
    )(x1t, x2p, pcl_feat, jf_p, rel, bmlp, w1t, w2t)

    y = pl.pallas_call(
        _bn_kernel,
        grid=(B,),
        in_specs=[
            pl.BlockSpec((1, N, D), lambda b: (b, 0, 0)),
            pl.BlockSpec((8, D), lambda b: (0, 0)),
            pl.BlockSpec((2, D), lambda b: (0, 0)),
        ],
        out_specs=pl.BlockSpec((1, N, D), lambda b: (b, 0, 0)),
        out_shape=jax.ShapeDtypeStruct((B, N, D), jnp.float32),
    )(x, stats, gb)
    return y
